# baseline (speedup 1.0000x reference)
import functools
import jax
import jax.numpy as jnp
from jax import lax
from jax.experimental import pallas as pl
from jax.experimental.pallas import tpu as pltpu
from jax.experimental.pallas import tpu_sc as plsc

D = 128
BOS = 98
L = 4
S = L + 1


def kernel(actions, action_table):
    B = actions.shape[0]
    NC, NS = 2, 16
    NW = NC * NS
    R = B * S // 128
    r_per_w = R // NW
    K = 1
    NBUF = 6
    n_step = (r_per_w + K - 1) // K

    table_aug = jnp.concatenate(
        [action_table, jnp.broadcast_to(action_table[BOS], (29, D))], axis=0
    )
    bos_ids = BOS + (jnp.arange(B, dtype=jnp.int32) % 30)
    fidx = jnp.concatenate(
        [bos_ids[:, None], actions.astype(jnp.int32)], axis=1
    ).reshape(NW, r_per_w, 128)

    mesh = plsc.VectorSubcoreMesh(core_axis_name="c", subcore_axis_name="s")

    @functools.partial(
        pl.kernel,
        out_type=jax.ShapeDtypeStruct((R, 128, D), jnp.float32),
        mesh=mesh,
        scratch_types=[
            pltpu.VMEM_SHARED((128, D), jnp.float32),
            pltpu.VMEM((1, r_per_w, 128), jnp.int32),
            [pltpu.VMEM((K, 128, D), jnp.float32) for _ in range(NBUF)],
            [pltpu.SemaphoreType.DMA for _ in range(NBUF)],
            [pltpu.SemaphoreType.DMA for _ in range(NBUF)],
        ],
    )
    def emb_kernel(fidx_hbm, table_hbm, out_hbm, tab_s, idx_v, bufs, gsems, wsems):
        sid = lax.axis_index("s")
        wid = sid * NC + lax.axis_index("c")
        row0 = wid * r_per_w

        @pl.when(sid == 0)
        def _():
            pltpu.sync_copy(table_hbm, tab_s)

        pltpu.sync_copy(fidx_hbm.at[pl.ds(wid, 1)], idx_v)
        plsc.subcore_barrier()

        def start_gathers(i, b):
            rows = min(K, r_per_w - i * K)
            return [
                pltpu.async_copy(
                    tab_s.at[idx_v.at[0, i * K + j]],
                    bufs[b].at[j],
                    gsems[b],
                )
                for j in range(rows)
            ]

        gathers = {i: start_gathers(i, i % NBUF) for i in range(min(NBUF - 1, n_step))}
        writes = {}
        for i in range(n_step):
            b = i % NBUF
            for cp in gathers.pop(i):
                cp.wait()
            rows = min(K, r_per_w - i * K)
            writes[i] = pltpu.async_copy(
                bufs[b].at[pl.ds(0, rows)],
                out_hbm.at[pl.ds(row0 + i * K, rows)],
                wsems[b],
            )
            j = i + NBUF - 1
            if j < n_step:
                if j >= NBUF:
                    writes.pop(j - NBUF).wait()
                gathers[j] = start_gathers(j, j % NBUF)
        for i in sorted(writes):
            writes.pop(i).wait()

    out = emb_kernel(fidx, table_aug)
    return out.reshape(B, S, D)


# speedup vs baseline: 1.0039x; 1.0039x over previous
import functools
import jax
import jax.numpy as jnp
from jax import lax
from jax.experimental import pallas as pl
from jax.experimental.pallas import tpu as pltpu
from jax.experimental.pallas import tpu_sc as plsc

D = 128
BOS = 98
L = 4
S = L + 1


def kernel(actions, action_table):
    B = actions.shape[0]
    NC, NS = 2, 16
    NW = NC * NS
    R = B * S // 128
    r_per_w = R // NW
    K = 1
    NBUF = 6
    n_step = (r_per_w + K - 1) // K

    table_aug = jnp.concatenate(
        [action_table, jnp.broadcast_to(action_table[BOS], (29, D))], axis=0
    )
    bos_ids = BOS + (jnp.arange(B, dtype=jnp.int32) % 30)
    fidx = jnp.concatenate(
        [bos_ids[:, None], actions.astype(jnp.int32)], axis=1
    ).reshape(NW, r_per_w, 128)

    mesh = plsc.VectorSubcoreMesh(core_axis_name="c", subcore_axis_name="s")

    @functools.partial(
        pl.kernel,
        out_type=jax.ShapeDtypeStruct((R, 128, D), jnp.float32),
        mesh=mesh,
        scratch_types=[
            pltpu.VMEM_SHARED((128, D), jnp.float32),
            pltpu.VMEM((1, r_per_w, 128), jnp.int32),
            [pltpu.VMEM((K, 128, D), jnp.float32) for _ in range(NBUF)],
            [pltpu.SemaphoreType.DMA for _ in range(NBUF)],
            [pltpu.SemaphoreType.DMA for _ in range(NBUF)],
        ],
    )
    def emb_kernel(fidx_hbm, table_hbm, out_hbm, tab_s, idx_v, bufs, gsems, wsems):
        sid = lax.axis_index("s")
        wid = sid * NC + lax.axis_index("c")
        row0 = wid * r_per_w

        @pl.when(sid == 0)
        def _():
            pltpu.sync_copy(table_hbm, tab_s)

        pltpu.sync_copy(fidx_hbm.at[pl.ds(wid, 1)], idx_v)
        plsc.subcore_barrier()

        def start_gathers(i, b):
            rows = min(K, r_per_w - i * K)
            return [
                pltpu.async_copy(
                    tab_s.at[idx_v.at[0, i * K + j]],
                    bufs[b].at[j],
                    gsems[b],
                )
                for j in range(rows)
            ]

        G = 3
        gathers = {i: start_gathers(i, i % NBUF) for i in range(min(G, n_step))}
        writes = {}
        for i in range(n_step):
            b = i % NBUF
            for cp in gathers.pop(i):
                cp.wait()
            rows = min(K, r_per_w - i * K)
            writes[i] = pltpu.async_copy(
                bufs[b].at[pl.ds(0, rows)],
                out_hbm.at[pl.ds(row0 + i * K, rows)],
                wsems[b],
            )
            j = i + G
            if j < n_step:
                if j >= NBUF:
                    writes.pop(j - NBUF).wait()
                gathers[j] = start_gathers(j, j % NBUF)
        for i in sorted(writes):
            writes.pop(i).wait()

    out = emb_kernel(fidx, table_aug)
    return out.reshape(B, S, D)
